# Initial kernel scaffold; baseline (speedup 1.0000x reference)
import jax
import jax.numpy as jnp
from jax.experimental import pallas as pl


def kernel(desc0, desc1, W0, att_src0, att_dst0, b0, mlp_W0, mlp_b0, W1, att_src1, att_dst1, b1, mlp_W1, mlp_b1):
    raise NotImplementedError("write your pallas kernel here")


# speedup vs baseline: 1511.4883x; 1511.4883x over previous
import functools

import jax
import jax.numpy as jnp
from jax.experimental import pallas as pl

B = 4
F = 128
S0 = 256
S1 = 256
H = 4
N = S0 + S1

_dotg = functools.partial(
    jax.lax.dot_general,
    precision=jax.lax.Precision.HIGHEST,
    preferred_element_type=jnp.float32,
)


def _dot(a, b):
    return _dotg(a, b, (((1,), (0,)), ((), ())))


def _dot_t(a, b):
    return _dotg(a, b, (((1,), (1,)), ((), ())))


def _lrelu(x):
    return jnp.where(x >= 0, x, 0.2 * x)


def _gat_body(x_ref,
              W0_ref, as0_ref, ad0_ref, b0_ref, mW0_ref, mb0_ref,
              W1_ref, as1_ref, ad1_ref, b1_ref, mW1_ref, mb1_ref,
              o_ref):
    x = x_ref[0]
    layers = (
        (W0_ref, as0_ref, ad0_ref, b0_ref, mW0_ref, mb0_ref, False),
        (W1_ref, as1_ref, ad1_ref, b1_ref, mW1_ref, mb1_ref, True),
    )
    for W_ref, as_ref, ad_ref, bias_ref, mW_ref, mb_ref, cross in layers:
        h = _dot(x, W_ref[...])
        msg_sets = []
        for s in (0, 1):
            dlo = s * S0
            slo = (1 - s) * S0 if cross else dlo
            acc = jnp.zeros((S0, F), jnp.float32)
            for hi in range(H):
                hs = h[slo:slo + S0, hi * F:(hi + 1) * F]
                hd = h[dlo:dlo + S0, hi * F:(hi + 1) * F]
                a_s = as_ref[hi:hi + 1, :]
                a_d = ad_ref[hi:hi + 1, :]
                row = _dot_t(a_s, hs)
                col = _dot_t(hd, a_d)
                sc = _lrelu(row + col)
                if cross:
                    s_self = _lrelu(_dot_t(hd, a_s) + col)
                    m = jnp.maximum(jnp.max(sc, axis=1, keepdims=True), s_self)
                    e = jnp.exp(sc - m)
                    e_self = jnp.exp(s_self - m)
                    den = jnp.sum(e, axis=1, keepdims=True) + e_self + 1e-16
                    acc = acc + _dot(e / den, hs) + (e_self / den) * hd
                else:
                    m = jnp.max(sc, axis=1, keepdims=True)
                    e = jnp.exp(sc - m)
                    den = jnp.sum(e, axis=1, keepdims=True) + 1e-16
                    acc = acc + _dot(e / den, hs)
            msg_sets.append(acc)
        msg1 = jnp.concatenate(msg_sets, axis=0)
        msg1 = msg1 * (1.0 / H) + bias_ref[...]
        msg1 = jnp.maximum(msg1, 0.0)
        mW = mW_ref[...]
        msg2 = _dot(x, mW[:F, :]) + _dot(msg1, mW[F:, :]) + mb_ref[...]
        x = x + msg2
    o_ref[0] = x


@jax.jit
def kernel(desc0, desc1, W0, att_src0, att_dst0, b0, mlp_W0, mlp_b0,
           W1, att_src1, att_dst1, b1, mlp_W1, mlp_b1):
    x = jnp.concatenate([desc0, desc1], axis=2)
    xin = jnp.transpose(x, (0, 2, 1)).astype(jnp.float32)

    full = lambda a: pl.BlockSpec(a.shape, lambda b: (0,) * a.ndim)
    args = (W0, att_src0, att_dst0, b0.reshape(1, F), mlp_W0,
            mlp_b0.reshape(1, F),
            W1, att_src1, att_dst1, b1.reshape(1, F), mlp_W1,
            mlp_b1.reshape(1, F))

    out = pl.pallas_call(
        _gat_body,
        grid=(B,),
        in_specs=[pl.BlockSpec((1, N, F), lambda b: (b, 0, 0))]
        + [full(a) for a in args],
        out_specs=pl.BlockSpec((1, N, F), lambda b: (b, 0, 0)),
        out_shape=jax.ShapeDtypeStruct((B, N, F), jnp.float32),
    )(xin, *args)

    xo = jnp.transpose(out, (0, 2, 1))
    return xo[:, :, :S0], xo[:, :, S0:]


# speedup vs baseline: 3898.5075x; 2.5793x over previous
import functools

import jax
import jax.numpy as jnp
from jax.experimental import pallas as pl

B = 4
F = 128
S0 = 256
S1 = 256
H = 4
N = S0 + S1

_dotg = functools.partial(
    jax.lax.dot_general,
    precision=jax.lax.Precision.DEFAULT,
    preferred_element_type=jnp.float32,
)


def _dot(a, b):
    return _dotg(a, b, (((1,), (0,)), ((), ())))


def _dot_t(a, b):
    return _dotg(a, b, (((1,), (1,)), ((), ())))


def _lrelu(x):
    return jnp.where(x >= 0, x, 0.2 * x)


def _gat_body(x_ref,
              W0_ref, as0_ref, ad0_ref, b0_ref, mW0_ref, mb0_ref,
              W1_ref, as1_ref, ad1_ref, b1_ref, mW1_ref, mb1_ref,
              o_ref):
    x = x_ref[0]
    layers = (
        (W0_ref, as0_ref, ad0_ref, b0_ref, mW0_ref, mb0_ref, False),
        (W1_ref, as1_ref, ad1_ref, b1_ref, mW1_ref, mb1_ref, True),
    )
    for W_ref, as_ref, ad_ref, bias_ref, mW_ref, mb_ref, cross in layers:
        h = _dot(x, W_ref[...])
        msg_sets = []
        for s in (0, 1):
            dlo = s * S0
            slo = (1 - s) * S0 if cross else dlo
            acc = jnp.zeros((S0, F), jnp.float32)
            for hi in range(H):
                hs = h[slo:slo + S0, hi * F:(hi + 1) * F]
                hd = h[dlo:dlo + S0, hi * F:(hi + 1) * F]
                a_s = as_ref[hi:hi + 1, :]
                a_d = ad_ref[hi:hi + 1, :]
                row = _dot_t(a_s, hs)
                col = _dot_t(hd, a_d)
                sc = _lrelu(row + col)
                if cross:
                    s_self = _lrelu(_dot_t(hd, a_s) + col)
                    m = jnp.maximum(jnp.max(sc, axis=1, keepdims=True), s_self)
                    e = jnp.exp(sc - m)
                    e_self = jnp.exp(s_self - m)
                    den = jnp.sum(e, axis=1, keepdims=True) + e_self + 1e-16
                    acc = acc + _dot(e / den, hs) + (e_self / den) * hd
                else:
                    m = jnp.max(sc, axis=1, keepdims=True)
                    e = jnp.exp(sc - m)
                    den = jnp.sum(e, axis=1, keepdims=True) + 1e-16
                    acc = acc + _dot(e / den, hs)
            msg_sets.append(acc)
        msg1 = jnp.concatenate(msg_sets, axis=0)
        msg1 = msg1 * (1.0 / H) + bias_ref[...]
        msg1 = jnp.maximum(msg1, 0.0)
        mW = mW_ref[...]
        msg2 = _dot(x, mW[:F, :]) + _dot(msg1, mW[F:, :]) + mb_ref[...]
        x = x + msg2
    o_ref[0] = x


@jax.jit
def kernel(desc0, desc1, W0, att_src0, att_dst0, b0, mlp_W0, mlp_b0,
           W1, att_src1, att_dst1, b1, mlp_W1, mlp_b1):
    x = jnp.concatenate([desc0, desc1], axis=2)
    xin = jnp.transpose(x, (0, 2, 1)).astype(jnp.float32)

    full = lambda a: pl.BlockSpec(a.shape, lambda b: (0,) * a.ndim)
    args = (W0, att_src0, att_dst0, b0.reshape(1, F), mlp_W0,
            mlp_b0.reshape(1, F),
            W1, att_src1, att_dst1, b1.reshape(1, F), mlp_W1,
            mlp_b1.reshape(1, F))

    out = pl.pallas_call(
        _gat_body,
        grid=(B,),
        in_specs=[pl.BlockSpec((1, N, F), lambda b: (b, 0, 0))]
        + [full(a) for a in args],
        out_specs=pl.BlockSpec((1, N, F), lambda b: (b, 0, 0)),
        out_shape=jax.ShapeDtypeStruct((B, N, F), jnp.float32),
    )(xin, *args)

    xo = jnp.transpose(out, (0, 2, 1))
    return xo[:, :, :S0], xo[:, :, S0:]
